# baseline (speedup 1.0000x reference)
import functools

import jax
import jax.numpy as jnp
from jax import lax
from jax.experimental import pallas as pl
from jax.experimental.pallas import tpu as pltpu
from jax.experimental.pallas import tpu_sc as plsc

B = 16384
D = 64
NEG = 5
NC = 2
NS = 16
NW = NC * NS
ROWS_PER_W = B // NW
C = 64
NCHUNK = ROWS_PER_W // C
L = 16
G = C // L
E_TOTAL = 1000000
EV = E_TOTAL // 2
WE = 256
NWIN = 3906
TAIL_E = E_TOTAL - NWIN * WE
TAIL_ROWS = TAIL_E // 2
WPW6 = 21

_SC_PARAMS = pltpu.CompilerParams(needs_layout_passes=False,
                                  use_tc_tiling_on_sc=True)
_MESH = plsc.VectorSubcoreMesh(core_axis_name="c", subcore_axis_name="s")


def _transpose_body(entT_hbm, tail_hbm, out_hbm,
                    in0, in1, in2, ot0, ot1,
                    s_i0, s_i1, s_i2, s_o0, s_o1):
    wid = lax.axis_index("s") * NC + lax.axis_index("c")
    iota = lax.iota(jnp.int32, L)
    ins = [in0, in1, in2]
    sis = [s_i0, s_i1, s_i2]
    ots = [ot0, ot1]
    sos = [s_o0, s_o1]

    @pl.when(wid == 0)
    def _():
        pltpu.sync_copy(tail_hbm, ot0.at[pl.ds(0, TAIL_ROWS)])
        pltpu.sync_copy(ot0.at[pl.ds(0, TAIL_ROWS)],
                        out_hbm.at[pl.ds(NWIN * (WE // 2), TAIL_ROWS)])

    def e0_of(wl):
        return pl.multiple_of((wl * NW + wid) * WE, WE)

    def o0_of(wl):
        return pl.multiple_of((wl * NW + wid) * (WE // 2), WE // 2)

    def start_in(wl, bi):
        @pl.when(wl * NW + wid < NWIN)
        def _():
            for fg in range(8):
                pltpu.async_copy(
                    entT_hbm.at[pl.ds(8 * fg, 8), pl.ds(e0_of(wl), WE)],
                    ins[bi].at[pl.ds(8 * fg, 8), :], sis[bi])

    def do_win(wl, bi, bo):
        ibuf, sem_i = ins[bi], sis[bi]
        obuf, sem_o = ots[bo], sos[bo]

        @pl.when(wl * NW + wid < NWIN)
        def _():
            for fg in range(8):
                pltpu.make_async_copy(
                    entT_hbm.at[pl.ds(8 * fg, 8), pl.ds(e0_of(wl), WE)],
                    ibuf.at[pl.ds(8 * fg, 8), :], sem_i).wait()

            @pl.when(wl >= 2)
            def _():
                pltpu.make_async_copy(
                    obuf, out_hbm.at[pl.ds(o0_of(wl - 2), WE // 2)],
                    sem_o).wait()

            @plsc.parallel_loop(0, WE // 2, 1, unroll=4)
            def row_body(k):
                for s in range(8):
                    dcol = iota + 16 * (s % 4)
                    ecol = jnp.full((L,), 2 * k + (s // 4), jnp.int32)
                    v = plsc.load_gather(ibuf, [dcol, ecol])
                    obuf[k, pl.ds(16 * s, L)] = v

            pltpu.async_copy(obuf,
                             out_hbm.at[pl.ds(o0_of(wl), WE // 2)],
                             sem_o)

    for p in range(3):
        start_in(p, p)

    def loop_body(i, carry):
        for u in range(6):
            wl = i * 6 + u
            do_win(wl, u % 3, u % 2)
            start_in(wl + 3, u % 3)
        return carry

    lax.fori_loop(0, WPW6, loop_body, 0)

    pltpu.make_async_copy(ot0, out_hbm.at[pl.ds(0, WE // 2)], s_o0).wait()
    pltpu.make_async_copy(ot1, out_hbm.at[pl.ds(0, WE // 2)], s_o1).wait()


_transpose_sc = functools.partial(
    pl.kernel,
    out_type=jax.ShapeDtypeStruct((EV, 2 * D), jnp.float32),
    mesh=_MESH,
    compiler_params=_SC_PARAMS,
    scratch_types=[
        pltpu.VMEM((D, WE), jnp.float32),
        pltpu.VMEM((D, WE), jnp.float32),
        pltpu.VMEM((D, WE), jnp.float32),
        pltpu.VMEM((WE // 2, 2 * D), jnp.float32),
        pltpu.VMEM((WE // 2, 2 * D), jnp.float32),
        pltpu.SemaphoreType.DMA,
        pltpu.SemaphoreType.DMA,
        pltpu.SemaphoreType.DMA,
        pltpu.SemaphoreType.DMA,
        pltpu.SemaphoreType.DMA,
    ],
)(_transpose_body)


def _transe_body(h_hbm, r_hbm, t_hbm, tneg_hbm, ent_hbm, rel_hbm,
                 pos_hbm, neg_hbm,
                 hidx, ridx, tidx, nraw, nidx,
                 hidx2, ridx2, tidx2, nidx2,
                 hbuf, rbuf, tbuf, nbuf,
                 pos_s, neg_s, sem):
    wid = lax.axis_index("s") * NC + lax.axis_index("c")
    wbase = wid * ROWS_PER_W

    def chunk_body(cc, carry):
        base = pl.multiple_of(wbase + cc * C, C)

        pltpu.sync_copy(h_hbm.at[pl.ds(base, C)], hidx)
        pltpu.sync_copy(r_hbm.at[pl.ds(base, C)], ridx)
        pltpu.sync_copy(t_hbm.at[pl.ds(base, C)], tidx)
        pltpu.sync_copy(tneg_hbm.at[pl.ds(base, C)], nraw)

        def regroup_body(g, carry2):
            rows = g * L + lax.iota(jnp.int32, L)
            hv = hidx[pl.ds(g * L, L)]
            rv = ridx[pl.ds(g * L, L)]
            tv = tidx[pl.ds(g * L, L)]
            hidx2[pl.ds(g * L, L)] = jnp.right_shift(hv, 1)
            ridx2[pl.ds(g * L, L)] = jnp.right_shift(rv, 1)
            tidx2[pl.ds(g * L, L)] = jnp.right_shift(tv, 1)
            for j in range(NEG):
                v = plsc.load_gather(nraw, [rows, jnp.full((L,), j, jnp.int32)])
                nidx[pl.ds(j * C + g * L, L)] = v
                nidx2[pl.ds(j * C + g * L, L)] = jnp.right_shift(v, 1)
            return carry2

        lax.fori_loop(0, G, regroup_body, 0)

        cps = [
            pltpu.async_copy(ent_hbm.at[hidx2], hbuf, sem),
            pltpu.async_copy(rel_hbm.at[ridx2], rbuf, sem),
            pltpu.async_copy(ent_hbm.at[tidx2], tbuf, sem),
        ]
        for j in range(NEG):
            cps.append(pltpu.async_copy(ent_hbm.at[nidx2.at[pl.ds(j * C, C)]],
                                        nbuf.at[pl.ds(j * C, C)], sem))
        for cp in cps:
            cp.wait()

        def group_body(g, carry2):
            rows = g * L + lax.iota(jnp.int32, L)
            rows_n = [rows + j * C for j in range(NEG)]
            half_h = jnp.left_shift(jnp.bitwise_and(hidx[pl.ds(g * L, L)], 1), 6)
            half_r = jnp.left_shift(jnp.bitwise_and(ridx[pl.ds(g * L, L)], 1), 6)
            half_t = jnp.left_shift(jnp.bitwise_and(tidx[pl.ds(g * L, L)], 1), 6)
            half_n = [
                jnp.left_shift(
                    jnp.bitwise_and(nidx[pl.ds(j * C + g * L, L)], 1), 6)
                for j in range(NEG)
            ]
            acc_p = jnp.zeros((L,), jnp.float32)
            acc_n = [jnp.zeros((L,), jnp.float32) for _ in range(NEG)]
            for d in range(D):
                hv = plsc.load_gather(hbuf, [rows, half_h + d])
                rv = plsc.load_gather(rbuf, [rows, half_r + d])
                tv = plsc.load_gather(tbuf, [rows, half_t + d])
                hr = hv + rv
                acc_p = acc_p + jnp.abs(hr - tv)
                for j in range(NEG):
                    nv = plsc.load_gather(nbuf, [rows_n[j], half_n[j] + d])
                    acc_n[j] = acc_n[j] + jnp.abs(hr - nv)
            pos_s[pl.ds(g * L, L)] = acc_p
            for j in range(NEG):
                plsc.store_scatter(neg_s, [rows, jnp.full((L,), j, jnp.int32)],
                                   acc_n[j])
            return carry2

        lax.fori_loop(0, G, group_body, 0)

        pltpu.sync_copy(pos_s, pos_hbm.at[pl.ds(base, C)])
        pltpu.sync_copy(neg_s, neg_hbm.at[pl.ds(base, C)])
        return carry

    lax.fori_loop(0, NCHUNK, chunk_body, 0)


_transe_sc = functools.partial(
    pl.kernel,
    out_type=[
        jax.ShapeDtypeStruct((B,), jnp.float32),
        jax.ShapeDtypeStruct((B, NEG), jnp.float32),
    ],
    mesh=_MESH,
    compiler_params=_SC_PARAMS,
    scratch_types=[
        pltpu.VMEM((C,), jnp.int32),
        pltpu.VMEM((C,), jnp.int32),
        pltpu.VMEM((C,), jnp.int32),
        pltpu.VMEM((C, NEG), jnp.int32),
        pltpu.VMEM((NEG * C,), jnp.int32),
        pltpu.VMEM((C,), jnp.int32),
        pltpu.VMEM((C,), jnp.int32),
        pltpu.VMEM((C,), jnp.int32),
        pltpu.VMEM((NEG * C,), jnp.int32),
        pltpu.VMEM((C, 2 * D), jnp.float32),
        pltpu.VMEM((C, 2 * D), jnp.float32),
        pltpu.VMEM((C, 2 * D), jnp.float32),
        pltpu.VMEM((NEG * C, 2 * D), jnp.float32),
        pltpu.VMEM((C,), jnp.float32),
        pltpu.VMEM((C, NEG), jnp.float32),
        pltpu.SemaphoreType.DMA,
    ],
)(_transe_body)


@jax.jit
def kernel(h, r, t, t_neg, entity_emb, relation_emb):
    h = h.astype(jnp.int32)
    r = r.astype(jnp.int32)
    t = t.astype(jnp.int32)
    t_neg = t_neg.astype(jnp.int32)
    ent_t = jnp.transpose(entity_emb)
    tail = jnp.reshape(entity_emb[NWIN * WE:, :], (TAIL_ROWS, 2 * D))
    ent2 = _transpose_sc(ent_t, tail)
    rel2 = jnp.reshape(relation_emb, (500, 2 * D))
    pos, neg = _transe_sc(h, r, t, t_neg, ent2, rel2)
    return pos, neg


# speedup vs baseline: 1.3063x; 1.3063x over previous
import functools

import jax
import jax.numpy as jnp
from jax import lax
from jax.experimental import pallas as pl
from jax.experimental.pallas import tpu as pltpu
from jax.experimental.pallas import tpu_sc as plsc

B = 16384
D = 64
NEG = 5
NC = 2
NS = 16
NW = NC * NS
ROWS_PER_W = B // NW
C = 128
NCHUNK = ROWS_PER_W // C
L = 16
G = C // L


def _transe_body(h_hbm, r_hbm, t_hbm, tneg_hbm, ent_hbm, rel_hbm,
                 pos_hbm, neg_hbm,
                 hidx, ridx, tidx, nidx,
                 hbuf, rbuf, tbuf, nbuf,
                 pos_s, neg_s, sem):
    wid = lax.axis_index("s") * NC + lax.axis_index("c")
    wbase = wid * ROWS_PER_W

    def chunk_body(cc, carry):
        base = pl.multiple_of(wbase + cc * C, C)

        pltpu.sync_copy(h_hbm.at[pl.ds(base, C)], hidx)
        pltpu.sync_copy(r_hbm.at[pl.ds(base, C)], ridx)
        pltpu.sync_copy(t_hbm.at[pl.ds(base, C)], tidx)
        for j in range(NEG):
            pltpu.sync_copy(tneg_hbm.at[pl.ds(j * B + base, C)],
                            nidx.at[pl.ds(j * C, C)])

        cps = [
            pltpu.async_copy(ent_hbm.at[hidx], hbuf, sem),
            pltpu.async_copy(rel_hbm.at[ridx], rbuf, sem),
            pltpu.async_copy(ent_hbm.at[tidx], tbuf, sem),
        ]
        for j in range(NEG):
            cps.append(pltpu.async_copy(ent_hbm.at[nidx.at[pl.ds(j * C, C)]],
                                        nbuf.at[pl.ds(j * C, C)], sem))
        for cp in cps:
            cp.wait()

        def group_body(g, carry2):
            rows = g * L + lax.iota(jnp.int32, L)
            rows_n = [rows + j * C for j in range(NEG)]
            acc_p = jnp.zeros((L,), jnp.float32)
            acc_n = [jnp.zeros((L,), jnp.float32) for _ in range(NEG)]
            for d in range(D):
                col = jnp.full((L,), d, jnp.int32)
                hv = plsc.load_gather(hbuf, [rows, col])
                rv = plsc.load_gather(rbuf, [rows, col])
                tv = plsc.load_gather(tbuf, [rows, col])
                hr = hv + rv
                acc_p = acc_p + jnp.abs(hr - tv)
                for j in range(NEG):
                    nv = plsc.load_gather(nbuf, [rows_n[j], col])
                    acc_n[j] = acc_n[j] + jnp.abs(hr - nv)
            pos_s[pl.ds(g * L, L)] = acc_p
            for j in range(NEG):
                neg_s[pl.ds(j * C + g * L, L)] = acc_n[j]
            return carry2

        lax.fori_loop(0, G, group_body, 0)

        pltpu.sync_copy(pos_s, pos_hbm.at[pl.ds(base, C)])
        for j in range(NEG):
            pltpu.sync_copy(neg_s.at[pl.ds(j * C, C)],
                            neg_hbm.at[pl.ds(j * B + base, C)])
        return carry

    lax.fori_loop(0, NCHUNK, chunk_body, 0)


_transe_sc = functools.partial(
    pl.kernel,
    out_type=[
        jax.ShapeDtypeStruct((B,), jnp.float32),
        jax.ShapeDtypeStruct((NEG * B,), jnp.float32),
    ],
    mesh=plsc.VectorSubcoreMesh(core_axis_name="c", subcore_axis_name="s"),
    compiler_params=pltpu.CompilerParams(needs_layout_passes=False,
                                         use_tc_tiling_on_sc=False),
    scratch_types=[
        pltpu.VMEM((C,), jnp.int32),
        pltpu.VMEM((C,), jnp.int32),
        pltpu.VMEM((C,), jnp.int32),
        pltpu.VMEM((NEG * C,), jnp.int32),
        pltpu.VMEM((C, D), jnp.float32),
        pltpu.VMEM((C, D), jnp.float32),
        pltpu.VMEM((C, D), jnp.float32),
        pltpu.VMEM((NEG * C, D), jnp.float32),
        pltpu.VMEM((C,), jnp.float32),
        pltpu.VMEM((NEG * C,), jnp.float32),
        pltpu.SemaphoreType.DMA,
    ],
)(_transe_body)


@jax.jit
def kernel(h, r, t, t_neg, entity_emb, relation_emb):
    h = h.astype(jnp.int32)
    r = r.astype(jnp.int32)
    t = t.astype(jnp.int32)
    tneg_t = jnp.transpose(t_neg.astype(jnp.int32)).reshape(NEG * B)
    pos, neg_flat = _transe_sc(h, r, t, tneg_t, entity_emb, relation_emb)
    neg = jnp.transpose(neg_flat.reshape(NEG, B))
    return pos, neg
